# baseline (speedup 1.0000x reference)
import jax
import jax.numpy as jnp
from jax.experimental import pallas as pl
from jax.experimental.pallas import tpu as pltpu

_B, _C, _K, _P = 16, 64, 1024, 1024


def _vq_kernel(x_ref, ea_ref, zq_ref, ind_ref, dsum_ref):
    ea = ea_ref[...]
    e = ea[:, :_C]
    esqh = 0.5 * jnp.sum(e * e, axis=1, keepdims=True)
    x = x_ref[0]
    s = jax.lax.dot_general(
        e, x, (((1,), (0,)), ((), ())),
        preferred_element_type=jnp.float32) - esqh
    smax = jnp.max(s, axis=0, keepdims=True)
    oh = (s == smax).astype(jnp.float32)
    out = jax.lax.dot_general(
        ea, oh, (((0,), (0,)), ((), ())),
        preferred_element_type=jnp.float32)
    r = 1.0 / out[_C + 2:_C + 3]
    zq_ref[0] = out[:_C] * r
    ind_f = (out[_C:_C + 1] * 32.0 + out[_C + 1:_C + 2]) * r
    ind_ref[0] = (ind_f + 0.5).astype(jnp.int32)
    d = jnp.sum(x * x) - 2.0 * jnp.sum(smax)
    dsum_ref[...] = jnp.full((1, 8, 128), d, jnp.float32)


def kernel(z_e, embed_weight):
    B, C, H, W = z_e.shape
    K = embed_weight.shape[0]
    P = H * W
    x = z_e.reshape(B, C, P)
    iota = jnp.arange(K, dtype=jnp.float32)[:, None]
    ea = jnp.concatenate(
        [embed_weight,
         jnp.floor(iota / 32.0),
         jnp.mod(iota, 32.0),
         jnp.ones((K, 1), jnp.float32)],
        axis=1)
    zq, ind3, dparts = pl.pallas_call(
        _vq_kernel,
        grid=(B,),
        in_specs=[
            pl.BlockSpec((1, C, P), lambda b: (b, 0, 0)),
            pl.BlockSpec((K, C + 3), lambda b: (0, 0)),
        ],
        out_specs=[
            pl.BlockSpec((1, C, P), lambda b: (b, 0, 0)),
            pl.BlockSpec((1, 1, P), lambda b: (b, 0, 0)),
            pl.BlockSpec((1, 8, 128), lambda b: (b, 0, 0)),
        ],
        out_shape=[
            jax.ShapeDtypeStruct((B, C, P), jnp.float32),
            jax.ShapeDtypeStruct((B, 1, P), jnp.int32),
            jax.ShapeDtypeStruct((B, 8, 128), jnp.float32),
        ],
    )(x, ea)
    z_q_out = zq.reshape(B, C, H, W)
    ind = ind3.reshape(B, H, W)
    diff = jnp.sum(dparts[:, 0, 0]) * (12.5 / (B * C * P))
    return (z_q_out, diff, ind)
